# baseline (speedup 1.0000x reference)
import functools

import jax
import jax.numpy as jnp
from jax import lax
from jax.experimental import pallas as pl
from jax.experimental.pallas import tpu as pltpu
from jax.experimental.pallas import tpu_sc as plsc

VOCAB = 32768
HIDDEN = 256
B, L = 64, 32
NTOK = B * L

VB = 1024


def _gather_sc(embed_table, flat_ids):
    info = plsc.get_sparse_core_info()
    nw = info.num_cores * info.num_subcores
    b_per_w = NTOK // nw
    mesh = plsc.VectorSubcoreMesh(core_axis_name="c", subcore_axis_name="s")

    @functools.partial(
        pl.kernel,
        out_type=jax.ShapeDtypeStruct((NTOK, HIDDEN), jnp.float32),
        mesh=mesh,
        scratch_types=[
            pltpu.VMEM((b_per_w,), jnp.int32),
            pltpu.VMEM((b_per_w, HIDDEN), jnp.float32),
            pltpu.SemaphoreType.DMA,
        ],
    )
    def gather_kernel(table_hbm, idx_hbm, out_hbm, idx_v, rows_v, sem):
        wid = lax.axis_index("s") * info.num_cores + lax.axis_index("c")
        base = wid * b_per_w
        pltpu.sync_copy(idx_hbm.at[pl.ds(base, b_per_w)], idx_v)
        pltpu.async_copy(table_hbm.at[idx_v], rows_v, sem).wait()
        pltpu.sync_copy(rows_v, out_hbm.at[pl.ds(base, b_per_w)])

    return gather_kernel(embed_table, flat_ids)


def _mm_kernel(h_ref, w_ref, out_ref):
    hb = h_ref[...].astype(jnp.bfloat16)
    wb = w_ref[...].astype(jnp.bfloat16)
    out_ref[...] = lax.dot_general(
        hb, wb, (((1,), (1,)), ((), ())), preferred_element_type=jnp.float32
    )


def _matmul_tc(h, head_w):
    return pl.pallas_call(
        _mm_kernel,
        grid=(VOCAB // VB,),
        in_specs=[
            pl.BlockSpec((NTOK, HIDDEN), lambda i: (0, 0)),
            pl.BlockSpec((VB, HIDDEN), lambda i: (i, 0)),
        ],
        out_specs=pl.BlockSpec((NTOK, VB), lambda i: (0, i)),
        out_shape=jax.ShapeDtypeStruct((NTOK, VOCAB), jnp.float32),
    )(h, head_w)


def kernel(input_ids, embed_table, head_w):
    flat_ids = input_ids.reshape(NTOK).astype(jnp.int32)
    h = _gather_sc(embed_table, flat_ids)
    logits = _matmul_tc(h, head_w)
    return logits.reshape(B, L, VOCAB)


# speedup vs baseline: 1.0204x; 1.0204x over previous
import functools

import jax
import jax.numpy as jnp
from jax import lax
from jax.experimental import pallas as pl
from jax.experimental.pallas import tpu as pltpu
from jax.experimental.pallas import tpu_sc as plsc

VOCAB = 32768
HIDDEN = 256
B, L = 64, 32
NTOK = B * L

VB = 2048


def _gather_sc(embed_table, flat_ids):
    info = plsc.get_sparse_core_info()
    nw = info.num_cores * info.num_subcores
    b_per_w = NTOK // nw
    mesh = plsc.VectorSubcoreMesh(core_axis_name="c", subcore_axis_name="s")

    @functools.partial(
        pl.kernel,
        out_type=jax.ShapeDtypeStruct((NTOK, HIDDEN), jnp.float32),
        mesh=mesh,
        scratch_types=[
            pltpu.VMEM((b_per_w,), jnp.int32),
            pltpu.VMEM((b_per_w, HIDDEN), jnp.float32),
            pltpu.SemaphoreType.DMA,
        ],
    )
    def gather_kernel(table_hbm, idx_hbm, out_hbm, idx_v, rows_v, sem):
        wid = lax.axis_index("s") * info.num_cores + lax.axis_index("c")
        base = wid * b_per_w
        pltpu.sync_copy(idx_hbm.at[pl.ds(base, b_per_w)], idx_v)
        pltpu.async_copy(table_hbm.at[idx_v], rows_v, sem).wait()
        pltpu.sync_copy(rows_v, out_hbm.at[pl.ds(base, b_per_w)])

    return gather_kernel(embed_table, flat_ids)


def _mm_kernel(h_ref, w_ref, out_ref):
    hb = h_ref[...].astype(jnp.bfloat16)
    wb = w_ref[...].astype(jnp.bfloat16)
    out_ref[...] = lax.dot_general(
        hb, wb, (((1,), (1,)), ((), ())), preferred_element_type=jnp.float32
    )


def _matmul_tc(h, head_w):
    return pl.pallas_call(
        _mm_kernel,
        grid=(VOCAB // VB,),
        in_specs=[
            pl.BlockSpec((NTOK, HIDDEN), lambda i: (0, 0)),
            pl.BlockSpec((VB, HIDDEN), lambda i: (i, 0)),
        ],
        out_specs=pl.BlockSpec((NTOK, VB), lambda i: (0, i)),
        out_shape=jax.ShapeDtypeStruct((NTOK, VOCAB), jnp.float32),
    )(h, head_w)


def kernel(input_ids, embed_table, head_w):
    flat_ids = input_ids.reshape(NTOK).astype(jnp.int32)
    h = _gather_sc(embed_table, flat_ids)
    logits = _matmul_tc(h, head_w)
    return logits.reshape(B, L, VOCAB)
